# baseline (speedup 1.0000x reference)
import functools

import jax
import jax.numpy as jnp
from jax import lax
from jax.experimental import pallas as pl
from jax.experimental.pallas import tpu as pltpu
from jax.experimental.pallas import tpu_sc as plsc

NODE_DIM = 128
EDGE_DIM = 16
N_NODES = 10000
N_EDGES = 320000

_BLK = 4000
_NB = N_EDGES // (8 * _BLK)

_HI = jax.lax.Precision.DEFAULT


def _proj_body(x_ref, wa_ref, wb_ref, p_ref, q_ref):
    x = x_ref[...]
    p_ref[...] = jnp.dot(x, wa_ref[...], precision=_HI,
                         preferred_element_type=jnp.float32)
    q_ref[...] = jnp.dot(x, wb_ref[...], precision=_HI,
                         preferred_element_type=jnp.float32)


def _project(node_feats, W1a, W1b):
    return pl.pallas_call(
        _proj_body,
        out_shape=[
            jax.ShapeDtypeStruct((N_NODES, EDGE_DIM), jnp.float32),
            jax.ShapeDtypeStruct((N_NODES, EDGE_DIM), jnp.float32),
        ],
    )(node_feats, W1a, W1b)


def _sc_gather(P, Q, idx8):
    info = plsc.get_sparse_core_info()
    NC, NS = info.num_cores, info.num_subcores
    NW = NC * NS
    CH = 1000
    NCH = N_EDGES // (NW * CH)
    PER_STRIPE = _BLK // CH

    mesh = plsc.VectorSubcoreMesh(core_axis_name="c", subcore_axis_name="s")

    @functools.partial(
        pl.kernel,
        mesh=mesh,
        out_type=jax.ShapeDtypeStruct((_NB, _BLK, 8, EDGE_DIM),
                                      jnp.float32),
        scratch_types=[
            pltpu.VMEM((CH,), jnp.int32),
            pltpu.VMEM((CH,), jnp.int32),
            pltpu.VMEM((CH,), jnp.int32),
            pltpu.VMEM((CH,), jnp.int32),
            pltpu.VMEM((CH, EDGE_DIM), jnp.float32),
            pltpu.VMEM((CH, EDGE_DIM), jnp.float32),
            pltpu.SemaphoreType.DMA,
            pltpu.SemaphoreType.DMA,
            pltpu.SemaphoreType.DMA,
            pltpu.SemaphoreType.DMA,
            pltpu.SemaphoreType.DMA,
            pltpu.SemaphoreType.DMA,
        ],
        compiler_params=pltpu.CompilerParams(use_tc_tiling_on_sc=False),
    )
    def body(p_hbm, q_hbm, idx_hbm, s_hbm,
             sva, dva, svb, dvb, ra, rb,
             spa, sqa, spb, sqb, swa, swb):
        wid = lax.axis_index("s") * NC + lax.axis_index("c")
        first = wid * NCH

        def coords(c):
            i = c // (8 * PER_STRIPE)
            rem = c - i * (8 * PER_STRIPE)
            g = rem // PER_STRIPE
            r0 = (rem - g * PER_STRIPE) * CH
            return i, g, r0

        def e_base(c):
            i, g, r0 = coords(c)
            return pl.multiple_of(i * (8 * _BLK) + g * _BLK + r0, 8)

        def idx_load(c, sv, dv, si):
            e0 = e_base(c)
            pltpu.async_copy(idx_hbm.at[0, pl.ds(e0, CH)], sv, si)
            pltpu.async_copy(idx_hbm.at[1, pl.ds(e0, CH)], dv, si)

        def drain_idx(sv, dv, si):
            pltpu.make_async_copy(idx_hbm.at[0, pl.ds(0, CH)], sv,
                                  si).wait()
            pltpu.make_async_copy(idx_hbm.at[1, pl.ds(0, CH)], dv,
                                  si).wait()

        def drain_p(r, sp):
            pltpu.make_async_copy(p_hbm.at[pl.ds(0, CH)], r, sp).wait()

        def drain_qadd(r, sq):
            pltpu.make_async_copy(q_hbm.at[pl.ds(0, CH)], r, sq).wait()

        def start_wb(c, r, sw):
            i, g, r0 = coords(c)
            pltpu.async_copy(r, s_hbm.at[i, pl.ds(r0, CH), g], sw)

        def drain_wb(c, r, sw):
            i, g, r0 = coords(c)
            pltpu.make_async_copy(
                r, s_hbm.at[i, pl.ds(r0, CH), g], sw).wait()

        idx_load(first, sva, dva, spa)
        idx_load(first + 1, svb, dvb, spb)
        drain_idx(sva, dva, spa)
        drain_idx(svb, dvb, spb)

        def pair(p, carry):
            ca = first + 2 * p
            cb = ca + 1

            @pl.when(p > 0)
            def _():
                drain_wb(ca - 2, ra, swa)
                drain_wb(cb - 2, rb, swb)
            pltpu.async_copy(p_hbm.at[sva], ra, spa)
            pltpu.async_copy(p_hbm.at[svb], rb, spb)
            drain_p(ra, spa)
            pltpu.async_copy(q_hbm.at[dva], ra, sqa, add=True)
            drain_p(rb, spb)
            pltpu.async_copy(q_hbm.at[dvb], rb, sqb, add=True)
            drain_qadd(ra, sqa)
            start_wb(ca, ra, swa)
            drain_qadd(rb, sqb)
            start_wb(cb, rb, swb)

            @pl.when(p < NCH // 2 - 1)
            def _():
                idx_load(ca + 2, sva, dva, spa)
                idx_load(cb + 2, svb, dvb, spb)
                drain_idx(sva, dva, spa)
                drain_idx(svb, dvb, spb)
            return carry

        lax.fori_loop(0, NCH // 2, pair, 0)
        last = first + NCH - 1
        drain_wb(last - 1, ra, swa)
        drain_wb(last, rb, swb)

    return body(P, Q, idx8)


def _edge_body(s_ref, et_ref, w1e_ref, b1_ref, w2_ref, b2_ref,
               ot_ref):
    st = s_ref[...].T
    s_t = jnp.concatenate(
        [st[EDGE_DIM * g:EDGE_DIM * (g + 1), :] for g in range(8)],
        axis=1)
    eterm = lax.dot_general(w1e_ref[...], et_ref[...],
                            (((0,), (0,)), ((), ())), precision=_HI,
                            preferred_element_type=jnp.float32)
    h = s_t + eterm + b1_ref[...][:, 0:1]
    h = jnp.where(h >= 0, h, 0.2 * h)
    ot = lax.dot_general(w2_ref[...], h,
                         (((0,), (0,)), ((), ())), precision=_HI,
                         preferred_element_type=jnp.float32)
    ot_ref[...] = ot + b2_ref[...][:, 0:1]


def _edge_mlp(sv, et, w1e, b1_col, w2, b2_col):
    rows = N_EDGES * EDGE_DIM // 128
    grid = rows // _BLK
    full = lambda i: (0, 0)
    blk = lambda i: (i, 0)
    lane_blk = lambda i: (0, i)
    return pl.pallas_call(
        _edge_body,
        grid=(grid,),
        in_specs=[
            pl.BlockSpec((_BLK, 128), blk),
            pl.BlockSpec((EDGE_DIM, _BLK * 8), lane_blk),
            pl.BlockSpec((EDGE_DIM, EDGE_DIM), full),
            pl.BlockSpec((EDGE_DIM, 128), full),
            pl.BlockSpec((EDGE_DIM, EDGE_DIM), full),
            pl.BlockSpec((EDGE_DIM, 128), full),
        ],
        out_specs=pl.BlockSpec((EDGE_DIM, _BLK * 8), lane_blk),
        out_shape=jax.ShapeDtypeStruct((EDGE_DIM, N_EDGES), jnp.float32),
    )(sv, et, w1e, b1_col, w2, b2_col)


def kernel(node_feats, edge_index, edge_feats, W1, b1, W2, b2):
    W1a = W1[:NODE_DIM]
    W1b = W1[NODE_DIM:2 * NODE_DIM]
    W1e = W1[2 * NODE_DIM:]

    P, Q = _project(node_feats, W1a, W1b)
    idx8 = jnp.pad(edge_index.astype(jnp.int32), ((0, 6), (0, 0)))
    S = _sc_gather(P, Q, idx8)

    b1_col = jnp.tile(b1[:, None], (1, 128))
    b2_col = jnp.tile(b2[:, None], (1, 128))

    rows = N_EDGES * EDGE_DIM // 128
    out_t = _edge_mlp(
        S.reshape(rows, 128),
        edge_feats.T,
        W1e, b1_col, W2, b2_col,
    )
    return out_t.T


# speedup vs baseline: 1.1262x; 1.1262x over previous
import functools

import jax
import jax.numpy as jnp
from jax import lax
from jax.experimental import pallas as pl
from jax.experimental.pallas import tpu as pltpu
from jax.experimental.pallas import tpu_sc as plsc

NODE_DIM = 128
EDGE_DIM = 16
N_NODES = 10000
N_EDGES = 320000

_BLK = 4000
_NB = N_EDGES // (8 * _BLK)

_HI = jax.lax.Precision.DEFAULT


def _proj_body(x_ref, wa_ref, wb_ref, ei_ref, p_ref, q_ref, s_ref, d_ref):
    x = x_ref[...]
    p_ref[...] = jnp.dot(x, wa_ref[...], precision=_HI,
                         preferred_element_type=jnp.float32)
    q_ref[...] = jnp.dot(x, wb_ref[...], precision=_HI,
                         preferred_element_type=jnp.float32)
    ei = ei_ref[...]
    s_ref[...] = ei[0]
    d_ref[...] = ei[1]


def _project(node_feats, W1a, W1b, edge_index):
    return pl.pallas_call(
        _proj_body,
        out_shape=[
            jax.ShapeDtypeStruct((N_NODES, EDGE_DIM), jnp.float32),
            jax.ShapeDtypeStruct((N_NODES, EDGE_DIM), jnp.float32),
            jax.ShapeDtypeStruct((N_EDGES,), jnp.int32),
            jax.ShapeDtypeStruct((N_EDGES,), jnp.int32),
        ],
    )(node_feats, W1a, W1b, edge_index)


def _sc_gather(P, Q, src, dst):
    info = plsc.get_sparse_core_info()
    NC, NS = info.num_cores, info.num_subcores
    NW = NC * NS
    CH = 1000
    NCH = N_EDGES // (NW * CH)
    PER_STRIPE = _BLK // CH

    mesh = plsc.VectorSubcoreMesh(core_axis_name="c", subcore_axis_name="s")

    @functools.partial(
        pl.kernel,
        mesh=mesh,
        out_type=jax.ShapeDtypeStruct((_NB, _BLK, 8, EDGE_DIM),
                                      jnp.float32),
        scratch_types=[
            pltpu.VMEM((CH,), jnp.int32),
            pltpu.VMEM((CH,), jnp.int32),
            pltpu.VMEM((CH,), jnp.int32),
            pltpu.VMEM((CH,), jnp.int32),
            pltpu.VMEM((CH, EDGE_DIM), jnp.float32),
            pltpu.VMEM((CH, EDGE_DIM), jnp.float32),
            pltpu.SemaphoreType.DMA,
            pltpu.SemaphoreType.DMA,
            pltpu.SemaphoreType.DMA,
            pltpu.SemaphoreType.DMA,
            pltpu.SemaphoreType.DMA,
            pltpu.SemaphoreType.DMA,
        ],
        compiler_params=pltpu.CompilerParams(use_tc_tiling_on_sc=False),
    )
    def body(p_hbm, q_hbm, src_hbm, dst_hbm, s_hbm,
             sva, dva, svb, dvb, ra, rb,
             spa, sqa, spb, sqb, swa, swb):
        wid = lax.axis_index("s") * NC + lax.axis_index("c")
        first = wid * NCH

        def coords(c):
            i = c // (8 * PER_STRIPE)
            rem = c - i * (8 * PER_STRIPE)
            g = rem // PER_STRIPE
            r0 = (rem - g * PER_STRIPE) * CH
            return i, g, r0

        def e_base(c):
            i, g, r0 = coords(c)
            return pl.multiple_of(i * (8 * _BLK) + g * _BLK + r0, 8)

        def idx_load(c, sv, dv, si):
            e0 = e_base(c)
            pltpu.async_copy(src_hbm.at[pl.ds(e0, CH)], sv, si)
            pltpu.async_copy(dst_hbm.at[pl.ds(e0, CH)], dv, si)

        def drain_idx(sv, dv, si):
            pltpu.make_async_copy(src_hbm.at[pl.ds(0, CH)], sv,
                                  si).wait()
            pltpu.make_async_copy(dst_hbm.at[pl.ds(0, CH)], dv,
                                  si).wait()

        def drain_p(r, sp):
            pltpu.make_async_copy(p_hbm.at[pl.ds(0, CH)], r, sp).wait()

        def drain_qadd(r, sq):
            pltpu.make_async_copy(q_hbm.at[pl.ds(0, CH)], r, sq).wait()

        def start_wb(c, r, sw):
            i, g, r0 = coords(c)
            pltpu.async_copy(r, s_hbm.at[i, pl.ds(r0, CH), g], sw)

        def drain_wb(c, r, sw):
            i, g, r0 = coords(c)
            pltpu.make_async_copy(
                r, s_hbm.at[i, pl.ds(r0, CH), g], sw).wait()

        idx_load(first, sva, dva, spa)
        idx_load(first + 1, svb, dvb, spb)
        drain_idx(sva, dva, spa)
        drain_idx(svb, dvb, spb)

        def pair(p, carry):
            ca = first + 2 * p
            cb = ca + 1

            @pl.when(p > 0)
            def _():
                drain_wb(ca - 2, ra, swa)
                drain_wb(cb - 2, rb, swb)
            pltpu.async_copy(p_hbm.at[sva], ra, spa)
            pltpu.async_copy(p_hbm.at[svb], rb, spb)
            drain_p(ra, spa)
            pltpu.async_copy(q_hbm.at[dva], ra, sqa, add=True)
            drain_p(rb, spb)
            pltpu.async_copy(q_hbm.at[dvb], rb, sqb, add=True)
            drain_qadd(ra, sqa)
            start_wb(ca, ra, swa)
            drain_qadd(rb, sqb)
            start_wb(cb, rb, swb)

            @pl.when(p < NCH // 2 - 1)
            def _():
                idx_load(ca + 2, sva, dva, spa)
                idx_load(cb + 2, svb, dvb, spb)
                drain_idx(sva, dva, spa)
                drain_idx(svb, dvb, spb)
            return carry

        lax.fori_loop(0, NCH // 2, pair, 0)
        last = first + NCH - 1
        drain_wb(last - 1, ra, swa)
        drain_wb(last, rb, swb)

    return body(P, Q, src, dst)


def _edge_body(s_ref, et_ref, w1e_ref, b1_ref, w2_ref, b2_ref,
               ot_ref):
    st = s_ref[...].T
    s_t = jnp.concatenate(
        [st[EDGE_DIM * g:EDGE_DIM * (g + 1), :] for g in range(8)],
        axis=1)
    eterm = lax.dot_general(w1e_ref[...], et_ref[...],
                            (((0,), (0,)), ((), ())), precision=_HI,
                            preferred_element_type=jnp.float32)
    h = s_t + eterm + b1_ref[...][:, 0:1]
    h = jnp.where(h >= 0, h, 0.2 * h)
    ot = lax.dot_general(w2_ref[...], h,
                         (((0,), (0,)), ((), ())), precision=_HI,
                         preferred_element_type=jnp.float32)
    ot_ref[...] = ot + b2_ref[...][:, 0:1]


def _edge_mlp(sv, et, w1e, b1_col, w2, b2_col):
    rows = N_EDGES * EDGE_DIM // 128
    grid = rows // _BLK
    full = lambda i: (0, 0)
    blk = lambda i: (i, 0)
    lane_blk = lambda i: (0, i)
    return pl.pallas_call(
        _edge_body,
        grid=(grid,),
        in_specs=[
            pl.BlockSpec((_BLK, 128), blk),
            pl.BlockSpec((EDGE_DIM, _BLK * 8), lane_blk),
            pl.BlockSpec((EDGE_DIM, EDGE_DIM), full),
            pl.BlockSpec((EDGE_DIM, 128), full),
            pl.BlockSpec((EDGE_DIM, EDGE_DIM), full),
            pl.BlockSpec((EDGE_DIM, 128), full),
        ],
        out_specs=pl.BlockSpec((EDGE_DIM, _BLK * 8), lane_blk),
        out_shape=jax.ShapeDtypeStruct((EDGE_DIM, N_EDGES), jnp.float32),
    )(sv, et, w1e, b1_col, w2, b2_col)


def kernel(node_feats, edge_index, edge_feats, W1, b1, W2, b2):
    W1a = W1[:NODE_DIM]
    W1b = W1[NODE_DIM:2 * NODE_DIM]
    W1e = W1[2 * NODE_DIM:]

    P, Q, src, dst = _project(node_feats, W1a, W1b,
                              edge_index.astype(jnp.int32))
    S = _sc_gather(P, Q, src, dst)

    b1_col = jnp.tile(b1[:, None], (1, 128))
    b2_col = jnp.tile(b2[:, None], (1, 128))

    rows = N_EDGES * EDGE_DIM // 128
    out_t = _edge_mlp(
        S.reshape(rows, 128),
        edge_feats.T,
        W1e, b1_col, W2, b2_col,
    )
    return out_t.T
